# initial kernel scaffold (unmeasured)
import jax
import jax.numpy as jnp
from jax import lax
from jax.experimental import pallas as pl
from jax.experimental.pallas import tpu as pltpu


def kernel(
    x,
):
    def body(*refs):
        pass

    out_shape = jax.ShapeDtypeStruct(..., jnp.float32)
    return pl.pallas_call(body, out_shape=out_shape)(...)


# baseline (device time: 17800 ns/iter reference)
import jax
import jax.numpy as jnp
from jax import lax
from jax.experimental import pallas as pl
from jax.experimental.pallas import tpu as pltpu

N_X = 2


def kernel(x):
    m_per, n = x.shape

    def body(x_ref, out_ref, send_sem, recv_sem):
        my_x = lax.axis_index("x")
        my_y = lax.axis_index("y")
        other_x = 1 - my_x

        out_ref[pl.ds(my_x * m_per, m_per), :] = x_ref[:, :].astype(jnp.bfloat16)

        barrier_sem = pltpu.get_barrier_semaphore()
        pl.semaphore_signal(
            barrier_sem,
            inc=1,
            device_id=(other_x, my_y),
            device_id_type=pl.DeviceIdType.MESH,
        )
        pl.semaphore_wait(barrier_sem, 1)

        rdma = pltpu.make_async_remote_copy(
            src_ref=out_ref.at[pl.ds(my_x * m_per, m_per), :],
            dst_ref=out_ref.at[pl.ds(my_x * m_per, m_per), :],
            send_sem=send_sem,
            recv_sem=recv_sem,
            device_id=(other_x, my_y),
            device_id_type=pl.DeviceIdType.MESH,
        )
        rdma.start()
        rdma.wait()

    return pl.pallas_call(
        body,
        out_shape=jax.ShapeDtypeStruct((N_X * m_per, n), jnp.bfloat16),
        in_specs=[pl.BlockSpec(memory_space=pltpu.VMEM)],
        out_specs=pl.BlockSpec(memory_space=pltpu.VMEM),
        scratch_shapes=[
            pltpu.SemaphoreType.DMA,
            pltpu.SemaphoreType.DMA,
        ],
        compiler_params=pltpu.CompilerParams(collective_id=0),
    )(x)


# device time: 15651 ns/iter; 1.1373x vs baseline; 1.1373x over previous
import jax
import jax.numpy as jnp
from jax import lax
from jax.experimental import pallas as pl
from jax.experimental.pallas import tpu as pltpu

N_X = 2
N_CHUNKS = 8


def kernel(x):
    m_per, n = x.shape
    half = m_per // 2
    ck = half // N_CHUNKS

    def body(x_ref, out_ref, x_send_sems, x_recv_sems, y_send_sems, y_recv_sems):
        my_x = lax.axis_index("x")
        my_y = lax.axis_index("y")
        other_x = 1 - my_x
        x_nbr = (other_x, my_y)
        y_nbr = (my_x, 1 - my_y)

        out_ref[pl.ds(my_x * m_per, m_per), :] = x_ref[:, :].astype(jnp.bfloat16)

        barrier_sem = pltpu.get_barrier_semaphore()
        for nbr in (x_nbr, y_nbr):
            pl.semaphore_signal(
                barrier_sem,
                inc=1,
                device_id=nbr,
                device_id_type=pl.DeviceIdType.MESH,
            )
        pl.semaphore_wait(barrier_sem, 2)

        x_rdmas = []
        for i in range(N_CHUNKS):
            r = pl.ds(my_x * m_per + my_y * half + i * ck, ck)
            rdma = pltpu.make_async_remote_copy(
                src_ref=out_ref.at[r, :],
                dst_ref=out_ref.at[r, :],
                send_sem=x_send_sems.at[i],
                recv_sem=x_recv_sems.at[i],
                device_id=x_nbr,
                device_id_type=pl.DeviceIdType.MESH,
            )
            rdma.start()
            x_rdmas.append(rdma)

        y_rdmas = []
        for i in range(N_CHUNKS):
            recv_r = pl.ds(other_x * m_per + my_y * half + i * ck, ck)
            recv = pltpu.make_async_remote_copy(
                src_ref=out_ref.at[recv_r, :],
                dst_ref=out_ref.at[recv_r, :],
                send_sem=x_send_sems.at[i],
                recv_sem=x_recv_sems.at[i],
                device_id=x_nbr,
                device_id_type=pl.DeviceIdType.MESH,
            )
            recv.wait_recv()
            fwd = pltpu.make_async_remote_copy(
                src_ref=out_ref.at[recv_r, :],
                dst_ref=out_ref.at[recv_r, :],
                send_sem=y_send_sems.at[i],
                recv_sem=y_recv_sems.at[i],
                device_id=y_nbr,
                device_id_type=pl.DeviceIdType.MESH,
            )
            fwd.start()
            y_rdmas.append(fwd)

        for i in range(N_CHUNKS):
            y_rdmas[i].wait_recv()
        for i in range(N_CHUNKS):
            x_rdmas[i].wait_send()
            y_rdmas[i].wait_send()

    return pl.pallas_call(
        body,
        out_shape=jax.ShapeDtypeStruct((N_X * m_per, n), jnp.bfloat16),
        in_specs=[pl.BlockSpec(memory_space=pltpu.VMEM)],
        out_specs=pl.BlockSpec(memory_space=pltpu.VMEM),
        scratch_shapes=[
            pltpu.SemaphoreType.DMA((N_CHUNKS,)),
            pltpu.SemaphoreType.DMA((N_CHUNKS,)),
            pltpu.SemaphoreType.DMA((N_CHUNKS,)),
            pltpu.SemaphoreType.DMA((N_CHUNKS,)),
        ],
        compiler_params=pltpu.CompilerParams(collective_id=0),
    )(x)


# device time: 14956 ns/iter; 1.1902x vs baseline; 1.0465x over previous
import jax
import jax.numpy as jnp
from jax import lax
from jax.experimental import pallas as pl
from jax.experimental.pallas import tpu as pltpu

N_X = 2
CK = 32
N_FWD = 13


def kernel(x):
    m_per, n = x.shape
    fwd_rows = N_FWD * CK
    direct_rows = m_per - fwd_rows
    tail_rows = direct_rows - fwd_rows

    def body(x_ref, out_ref, x_send_sems, x_recv_sems, y_send_sems, y_recv_sems):
        my_x = lax.axis_index("x")
        my_y = lax.axis_index("y")
        other_x = 1 - my_x
        x_nbr = (other_x, my_y)
        y_nbr = (my_x, 1 - my_y)

        base1 = my_y * direct_rows
        base2 = fwd_rows

        out_ref[pl.ds(my_x * m_per, m_per), :] = x_ref[:, :].astype(jnp.bfloat16)

        barrier_sem = pltpu.get_barrier_semaphore()
        for nbr in (x_nbr, y_nbr):
            pl.semaphore_signal(
                barrier_sem,
                inc=1,
                device_id=nbr,
                device_id_type=pl.DeviceIdType.MESH,
            )
        pl.semaphore_wait(barrier_sem, 2)

        x_rdmas = []
        for i in range(N_FWD):
            r = pl.ds(my_x * m_per + base1 + i * CK, CK)
            rdma = pltpu.make_async_remote_copy(
                src_ref=out_ref.at[r, :],
                dst_ref=out_ref.at[r, :],
                send_sem=x_send_sems.at[i],
                recv_sem=x_recv_sems.at[i],
                device_id=x_nbr,
                device_id_type=pl.DeviceIdType.MESH,
            )
            rdma.start()
            x_rdmas.append(rdma)
        tail = pltpu.make_async_remote_copy(
            src_ref=out_ref.at[pl.ds(my_x * m_per + base2, tail_rows), :],
            dst_ref=out_ref.at[pl.ds(my_x * m_per + base2, tail_rows), :],
            send_sem=x_send_sems.at[N_FWD],
            recv_sem=x_recv_sems.at[N_FWD],
            device_id=x_nbr,
            device_id_type=pl.DeviceIdType.MESH,
        )
        tail.start()

        y_rdmas = []
        for i in range(N_FWD):
            recv_r = pl.ds(other_x * m_per + base1 + i * CK, CK)
            recv = pltpu.make_async_remote_copy(
                src_ref=out_ref.at[recv_r, :],
                dst_ref=out_ref.at[recv_r, :],
                send_sem=x_send_sems.at[i],
                recv_sem=x_recv_sems.at[i],
                device_id=x_nbr,
                device_id_type=pl.DeviceIdType.MESH,
            )
            recv.wait_recv()
            fwd = pltpu.make_async_remote_copy(
                src_ref=out_ref.at[recv_r, :],
                dst_ref=out_ref.at[recv_r, :],
                send_sem=y_send_sems.at[i],
                recv_sem=y_recv_sems.at[i],
                device_id=y_nbr,
                device_id_type=pl.DeviceIdType.MESH,
            )
            fwd.start()
            y_rdmas.append(fwd)

        tail.wait_recv()
        for i in range(N_FWD):
            y_rdmas[i].wait_recv()
        for i in range(N_FWD):
            x_rdmas[i].wait_send()
            y_rdmas[i].wait_send()
        tail.wait_send()

    return pl.pallas_call(
        body,
        out_shape=jax.ShapeDtypeStruct((N_X * m_per, n), jnp.bfloat16),
        in_specs=[pl.BlockSpec(memory_space=pltpu.VMEM)],
        out_specs=pl.BlockSpec(memory_space=pltpu.VMEM),
        scratch_shapes=[
            pltpu.SemaphoreType.DMA((N_FWD + 1,)),
            pltpu.SemaphoreType.DMA((N_FWD + 1,)),
            pltpu.SemaphoreType.DMA((N_FWD,)),
            pltpu.SemaphoreType.DMA((N_FWD,)),
        ],
        compiler_params=pltpu.CompilerParams(collective_id=0),
    )(x)


# device time: 14859 ns/iter; 1.1979x vs baseline; 1.0065x over previous
import jax
import jax.numpy as jnp
from jax import lax
from jax.experimental import pallas as pl
from jax.experimental.pallas import tpu as pltpu

N_X = 2
CK = 32
N_FWD = 13


def kernel(x):
    m_per, n = x.shape
    fwd_rows = N_FWD * CK
    direct_rows = m_per - fwd_rows
    tail_rows = direct_rows - fwd_rows

    def body(x_ref, out_ref, x_send_sems, x_recv_sems, y_send_sems, y_recv_sems):
        my_x = lax.axis_index("x")
        my_y = lax.axis_index("y")
        other_x = 1 - my_x
        x_nbr = (other_x, my_y)
        y_nbr = (my_x, 1 - my_y)

        base1 = my_y * direct_rows
        base2 = fwd_rows

        barrier_sem = pltpu.get_barrier_semaphore()
        for nbr in (x_nbr, y_nbr):
            pl.semaphore_signal(
                barrier_sem,
                inc=1,
                device_id=nbr,
                device_id_type=pl.DeviceIdType.MESH,
            )

        out_ref[pl.ds(my_x * m_per + base1, fwd_rows), :] = x_ref[
            pl.ds(base1, fwd_rows), :
        ].astype(jnp.bfloat16)
        out_ref[pl.ds(my_x * m_per + fwd_rows, tail_rows), :] = x_ref[
            pl.ds(fwd_rows, tail_rows), :
        ].astype(jnp.bfloat16)

        pl.semaphore_wait(barrier_sem, 2)

        x_rdmas = []
        for i in range(N_FWD):
            r = pl.ds(my_x * m_per + base1 + i * CK, CK)
            rdma = pltpu.make_async_remote_copy(
                src_ref=out_ref.at[r, :],
                dst_ref=out_ref.at[r, :],
                send_sem=x_send_sems.at[i],
                recv_sem=x_recv_sems.at[i],
                device_id=x_nbr,
                device_id_type=pl.DeviceIdType.MESH,
            )
            rdma.start()
            x_rdmas.append(rdma)
        tail = pltpu.make_async_remote_copy(
            src_ref=out_ref.at[pl.ds(my_x * m_per + base2, tail_rows), :],
            dst_ref=out_ref.at[pl.ds(my_x * m_per + base2, tail_rows), :],
            send_sem=x_send_sems.at[N_FWD],
            recv_sem=x_recv_sems.at[N_FWD],
            device_id=x_nbr,
            device_id_type=pl.DeviceIdType.MESH,
        )
        tail.start()

        rest = (1 - my_y) * direct_rows
        out_ref[pl.ds(my_x * m_per + rest, fwd_rows), :] = x_ref[
            pl.ds(rest, fwd_rows), :
        ].astype(jnp.bfloat16)

        y_rdmas = []
        for i in range(N_FWD):
            recv_r = pl.ds(other_x * m_per + base1 + i * CK, CK)
            recv = pltpu.make_async_remote_copy(
                src_ref=out_ref.at[recv_r, :],
                dst_ref=out_ref.at[recv_r, :],
                send_sem=x_send_sems.at[i],
                recv_sem=x_recv_sems.at[i],
                device_id=x_nbr,
                device_id_type=pl.DeviceIdType.MESH,
            )
            recv.wait_recv()
            fwd = pltpu.make_async_remote_copy(
                src_ref=out_ref.at[recv_r, :],
                dst_ref=out_ref.at[recv_r, :],
                send_sem=y_send_sems.at[i],
                recv_sem=y_recv_sems.at[i],
                device_id=y_nbr,
                device_id_type=pl.DeviceIdType.MESH,
            )
            fwd.start()
            y_rdmas.append(fwd)

        tail.wait_recv()
        for i in range(N_FWD):
            y_rdmas[i].wait_recv()
        for i in range(N_FWD):
            x_rdmas[i].wait_send()
            y_rdmas[i].wait_send()
        tail.wait_send()

    return pl.pallas_call(
        body,
        out_shape=jax.ShapeDtypeStruct((N_X * m_per, n), jnp.bfloat16),
        in_specs=[pl.BlockSpec(memory_space=pltpu.VMEM)],
        out_specs=pl.BlockSpec(memory_space=pltpu.VMEM),
        scratch_shapes=[
            pltpu.SemaphoreType.DMA((N_FWD + 1,)),
            pltpu.SemaphoreType.DMA((N_FWD + 1,)),
            pltpu.SemaphoreType.DMA((N_FWD,)),
            pltpu.SemaphoreType.DMA((N_FWD,)),
        ],
        compiler_params=pltpu.CompilerParams(collective_id=0),
    )(x)


# device time: 14700 ns/iter; 1.2109x vs baseline; 1.0108x over previous
import jax
import jax.numpy as jnp
from jax import lax
from jax.experimental import pallas as pl
from jax.experimental.pallas import tpu as pltpu

N_X = 2
CK = 32
N_FWD = 14


def kernel(x):
    m_per, n = x.shape
    fwd_rows = N_FWD * CK
    direct_rows = m_per - fwd_rows
    tail_rows = direct_rows - fwd_rows

    def body(x_ref, out_ref, x_send_sems, x_recv_sems, y_send_sems, y_recv_sems):
        my_x = lax.axis_index("x")
        my_y = lax.axis_index("y")
        other_x = 1 - my_x
        x_nbr = (other_x, my_y)
        y_nbr = (my_x, 1 - my_y)

        base1 = my_y * direct_rows
        base2 = fwd_rows

        barrier_sem = pltpu.get_barrier_semaphore()
        for nbr in (x_nbr, y_nbr):
            pl.semaphore_signal(
                barrier_sem,
                inc=1,
                device_id=nbr,
                device_id_type=pl.DeviceIdType.MESH,
            )

        out_ref[pl.ds(my_x * m_per + base1, fwd_rows), :] = x_ref[
            pl.ds(base1, fwd_rows), :
        ].astype(jnp.bfloat16)
        out_ref[pl.ds(my_x * m_per + fwd_rows, tail_rows), :] = x_ref[
            pl.ds(fwd_rows, tail_rows), :
        ].astype(jnp.bfloat16)

        pl.semaphore_wait(barrier_sem, 2)

        x_rdmas = []
        for i in range(N_FWD):
            r = pl.ds(my_x * m_per + base1 + i * CK, CK)
            rdma = pltpu.make_async_remote_copy(
                src_ref=out_ref.at[r, :],
                dst_ref=out_ref.at[r, :],
                send_sem=x_send_sems.at[i],
                recv_sem=x_recv_sems.at[i],
                device_id=x_nbr,
                device_id_type=pl.DeviceIdType.MESH,
            )
            rdma.start()
            x_rdmas.append(rdma)
        tail = pltpu.make_async_remote_copy(
            src_ref=out_ref.at[pl.ds(my_x * m_per + base2, tail_rows), :],
            dst_ref=out_ref.at[pl.ds(my_x * m_per + base2, tail_rows), :],
            send_sem=x_send_sems.at[N_FWD],
            recv_sem=x_recv_sems.at[N_FWD],
            device_id=x_nbr,
            device_id_type=pl.DeviceIdType.MESH,
        )
        tail.start()

        rest = (1 - my_y) * direct_rows
        out_ref[pl.ds(my_x * m_per + rest, fwd_rows), :] = x_ref[
            pl.ds(rest, fwd_rows), :
        ].astype(jnp.bfloat16)

        y_rdmas = []
        for i in range(N_FWD):
            recv_r = pl.ds(other_x * m_per + base1 + i * CK, CK)
            recv = pltpu.make_async_remote_copy(
                src_ref=out_ref.at[recv_r, :],
                dst_ref=out_ref.at[recv_r, :],
                send_sem=x_send_sems.at[i],
                recv_sem=x_recv_sems.at[i],
                device_id=x_nbr,
                device_id_type=pl.DeviceIdType.MESH,
            )
            recv.wait_recv()
            fwd = pltpu.make_async_remote_copy(
                src_ref=out_ref.at[recv_r, :],
                dst_ref=out_ref.at[recv_r, :],
                send_sem=y_send_sems.at[i],
                recv_sem=y_recv_sems.at[i],
                device_id=y_nbr,
                device_id_type=pl.DeviceIdType.MESH,
            )
            fwd.start()
            y_rdmas.append(fwd)

        tail.wait_recv()
        for i in range(N_FWD):
            y_rdmas[i].wait_recv()
        for i in range(N_FWD):
            x_rdmas[i].wait_send()
            y_rdmas[i].wait_send()
        tail.wait_send()

    return pl.pallas_call(
        body,
        out_shape=jax.ShapeDtypeStruct((N_X * m_per, n), jnp.bfloat16),
        in_specs=[pl.BlockSpec(memory_space=pltpu.VMEM)],
        out_specs=pl.BlockSpec(memory_space=pltpu.VMEM),
        scratch_shapes=[
            pltpu.SemaphoreType.DMA((N_FWD + 1,)),
            pltpu.SemaphoreType.DMA((N_FWD + 1,)),
            pltpu.SemaphoreType.DMA((N_FWD,)),
            pltpu.SemaphoreType.DMA((N_FWD,)),
        ],
        compiler_params=pltpu.CompilerParams(collective_id=0),
    )(x)
